# Initial kernel scaffold; baseline (speedup 1.0000x reference)
import jax
import jax.numpy as jnp
from jax.experimental import pallas as pl


def kernel(x, emb_e, emb_r):
    raise NotImplementedError("write your pallas kernel here")


# speedup vs baseline: 2.0402x; 2.0402x over previous
import functools

import jax
import jax.numpy as jnp
from jax import lax
from jax.experimental import pallas as pl
from jax.experimental.pallas import tpu as pltpu
from jax.experimental.pallas import tpu_sc as plsc

_BS = 16384
_EMB = 128
_CHUNK = 128
_NC = 2
_NS = 16
_NW = _NC * _NS
_NROWS = _BS // _CHUNK
_CPW = _NROWS // _NW

_mesh = plsc.VectorSubcoreMesh(core_axis_name="c", subcore_axis_name="s")


@functools.partial(
    pl.kernel,
    mesh=_mesh,
    out_type=(
        jax.ShapeDtypeStruct((_BS, _EMB), jnp.float32),
        jax.ShapeDtypeStruct((_BS, _EMB), jnp.float32),
        jax.ShapeDtypeStruct((_BS, _EMB), jnp.float32),
    ),
    scratch_types=[
        pltpu.VMEM((3, _CHUNK), jnp.int32),
        pltpu.VMEM((_CHUNK, _EMB), jnp.float32),
        pltpu.SemaphoreType.DMA,
    ],
)
def _lookup(h_hbm, r_hbm, t_hbm, emb_e_hbm, emb_r_hbm,
            out_h, out_r, out_t, idx_v, rows_v, sem):
    wid = lax.axis_index("s") * _NC + lax.axis_index("c")
    for j in range(_CPW):
        row = wid * _CPW + j
        base = row * _CHUNK
        for slot, (idx_hbm, table, out) in enumerate((
                (h_hbm, emb_e_hbm, out_h),
                (r_hbm, emb_r_hbm, out_r),
                (t_hbm, emb_e_hbm, out_t))):
            pltpu.sync_copy(idx_hbm.at[row], idx_v.at[slot])
            pltpu.async_copy(table.at[idx_v.at[slot]], rows_v, sem).wait()
            pltpu.sync_copy(rows_v, out.at[pl.ds(base, _CHUNK)])


def kernel(x, emb_e, emb_r):
    h = x[:, 0].reshape(_NROWS, _CHUNK)
    r = x[:, 1].reshape(_NROWS, _CHUNK)
    t = x[:, 2].reshape(_NROWS, _CHUNK)
    return _lookup(h, r, t, emb_e, emb_r)


# speedup vs baseline: 2.5559x; 1.2528x over previous
import functools

import jax
import jax.numpy as jnp
from jax import lax
from jax.experimental import pallas as pl
from jax.experimental.pallas import tpu as pltpu
from jax.experimental.pallas import tpu_sc as plsc

_BS = 16384
_EMB = 128
_CHUNK = 128
_NC = 2
_NS = 16
_NW = _NC * _NS
_NROWS = _BS // _CHUNK
_CPW = _NROWS // _NW
_NTASK = 3 * _CPW
_NBUF = 4

_mesh = plsc.VectorSubcoreMesh(core_axis_name="c", subcore_axis_name="s")


@functools.partial(
    pl.kernel,
    mesh=_mesh,
    out_type=(
        jax.ShapeDtypeStruct((_BS, _EMB), jnp.float32),
        jax.ShapeDtypeStruct((_BS, _EMB), jnp.float32),
        jax.ShapeDtypeStruct((_BS, _EMB), jnp.float32),
    ),
    scratch_types=(
        [pltpu.VMEM((_NTASK, _CHUNK), jnp.int32),
         pltpu.VMEM((_NBUF, _CHUNK, _EMB), jnp.float32)]
        + [pltpu.SemaphoreType.DMA] * (2 * _NBUF)
    ),
)
def _lookup(h_hbm, r_hbm, t_hbm, emb_e_hbm, emb_r_hbm,
            out_h, out_r, out_t, idx_v, rows_v, *sems):
    gsem, ssem = sems[:_NBUF], sems[_NBUF:]
    wid = lax.axis_index("s") * _NC + lax.axis_index("c")
    c0 = wid * _CPW

    pltpu.sync_copy(h_hbm.at[pl.ds(c0, _CPW)], idx_v.at[pl.ds(0, _CPW)])
    pltpu.sync_copy(r_hbm.at[pl.ds(c0, _CPW)], idx_v.at[pl.ds(_CPW, _CPW)])
    pltpu.sync_copy(t_hbm.at[pl.ds(c0, _CPW)], idx_v.at[pl.ds(2 * _CPW, _CPW)])

    tasks = []
    for s, (table, out) in enumerate(
            ((emb_e_hbm, out_h), (emb_r_hbm, out_r), (emb_e_hbm, out_t))):
        for j in range(_CPW):
            tasks.append((s * _CPW + j, table, out, (c0 + j) * _CHUNK))

    def fire_gather(i):
        slot, table, _, _ = tasks[i]
        return pltpu.async_copy(
            table.at[idx_v.at[slot]], rows_v.at[i % _NBUF], gsem[i % _NBUF])

    g_desc = [fire_gather(i) for i in range(_NBUF)] + [None] * (_NTASK - _NBUF)
    s_desc = [None] * _NTASK
    for i in range(_NTASK):
        _, _, out, obase = tasks[i]
        b = i % _NBUF
        g_desc[i].wait()
        s_desc[i] = pltpu.async_copy(
            rows_v.at[b], out.at[pl.ds(obase, _CHUNK)], ssem[b])
        if i + _NBUF < _NTASK:
            s_desc[i].wait()
            g_desc[i + _NBUF] = fire_gather(i + _NBUF)
    for i in range(_NTASK - _NBUF, _NTASK):
        s_desc[i].wait()


def kernel(x, emb_e, emb_r):
    h = x[:, 0].reshape(_NROWS, _CHUNK)
    r = x[:, 1].reshape(_NROWS, _CHUNK)
    t = x[:, 2].reshape(_NROWS, _CHUNK)
    return _lookup(h, r, t, emb_e, emb_r)


# speedup vs baseline: 2.6140x; 1.0227x over previous
import functools

import jax
import jax.numpy as jnp
from jax import lax
from jax.experimental import pallas as pl
from jax.experimental.pallas import tpu as pltpu
from jax.experimental.pallas import tpu_sc as plsc

_BS = 16384
_EMB = 128
_CHUNK = 128
_NC = 2
_NS = 16
_NW = _NC * _NS
_NROWS = _BS // _CHUNK
_CPW = _NROWS // _NW
_NTASK = 3 * _CPW
_NBUF = 6

_mesh = plsc.VectorSubcoreMesh(core_axis_name="c", subcore_axis_name="s")


@functools.partial(
    pl.kernel,
    mesh=_mesh,
    out_type=(
        jax.ShapeDtypeStruct((_BS, _EMB), jnp.float32),
        jax.ShapeDtypeStruct((_BS, _EMB), jnp.float32),
        jax.ShapeDtypeStruct((_BS, _EMB), jnp.float32),
    ),
    scratch_types=(
        [pltpu.VMEM((_NTASK, _CHUNK), jnp.int32),
         pltpu.VMEM((_NBUF, _CHUNK, _EMB), jnp.float32)]
        + [pltpu.SemaphoreType.DMA] * (2 * _NBUF)
    ),
)
def _lookup(h_hbm, r_hbm, t_hbm, emb_e_hbm, emb_r_hbm,
            out_h, out_r, out_t, idx_v, rows_v, *sems):
    gsem, ssem = sems[:_NBUF], sems[_NBUF:]
    wid = lax.axis_index("s") * _NC + lax.axis_index("c")
    c0 = wid * _CPW

    pltpu.sync_copy(h_hbm.at[pl.ds(c0, _CPW)], idx_v.at[pl.ds(0, _CPW)])
    pltpu.sync_copy(r_hbm.at[pl.ds(c0, _CPW)], idx_v.at[pl.ds(_CPW, _CPW)])
    pltpu.sync_copy(t_hbm.at[pl.ds(c0, _CPW)], idx_v.at[pl.ds(2 * _CPW, _CPW)])

    tasks = []
    for s, (table, out) in enumerate(
            ((emb_e_hbm, out_h), (emb_r_hbm, out_r), (emb_e_hbm, out_t))):
        for j in range(_CPW):
            tasks.append((s * _CPW + j, table, out, (c0 + j) * _CHUNK))

    def fire_gather(i):
        slot, table, _, _ = tasks[i]
        return pltpu.async_copy(
            table.at[idx_v.at[slot]], rows_v.at[i % _NBUF], gsem[i % _NBUF])

    g_desc = [fire_gather(i) for i in range(_NBUF)] + [None] * (_NTASK - _NBUF)
    s_desc = [None] * _NTASK
    for i in range(_NTASK):
        _, _, out, obase = tasks[i]
        b = i % _NBUF
        g_desc[i].wait()
        s_desc[i] = pltpu.async_copy(
            rows_v.at[b], out.at[pl.ds(obase, _CHUNK)], ssem[b])
        if i + _NBUF < _NTASK:
            s_desc[i].wait()
            g_desc[i + _NBUF] = fire_gather(i + _NBUF)
    for i in range(_NTASK - _NBUF, _NTASK):
        s_desc[i].wait()


def kernel(x, emb_e, emb_r):
    h = x[:, 0].reshape(_NROWS, _CHUNK)
    r = x[:, 1].reshape(_NROWS, _CHUNK)
    t = x[:, 2].reshape(_NROWS, _CHUNK)
    return _lookup(h, r, t, emb_e, emb_r)
